# baseline (speedup 1.0000x reference)
import functools

import jax
import jax.numpy as jnp
from jax import lax
from jax.experimental import pallas as pl
from jax.experimental.pallas import tpu as pltpu
from jax.experimental.pallas import tpu_sc as plsc

_E_SIZE = 6884
_NE = 320000
_NR = 990
_D = 128
_L = 16

_NC = 2
_NS = 16
_NW = _NC * _NS
_EPW = _NE // _NW
_B = 80
_NCHUNK = _EPW // _B
_NT = 4


def _sqrt_nr(x):
    i = lax.bitcast_convert_type(x, jnp.int32)
    y = lax.bitcast_convert_type(jnp.int32(0x5F3759DF) - (i >> 1), jnp.float32)
    for _ in range(2):
        y = y * (1.5 - 0.5 * x * y * y)
    return x * y


def _tile_body(node_hbm, rel_hbm, src_hbm, tgt_hbm, nsrc_hbm, ntgt_hbm, et_hbm,
               out_hbm,
               rel_v, idx_v, rows_v, acc_v, semr0, semr1, semi0, semi1):
    wid = lax.axis_index("s") * _NC + lax.axis_index("c")
    base = pl.multiple_of(wid * _EPW, 8)

    pltpu.sync_copy(rel_hbm, rel_v)

    semr = (semr0, semr1)
    semi = (semi0, semi1)
    idx_hbms = (src_hbm, tgt_hbm, nsrc_hbm, ntgt_hbm, et_hbm)

    def fire_idx(c, p):
        off = pl.multiple_of(base + c * _B, 8)
        for j, h in enumerate(idx_hbms):
            pltpu.async_copy(h.at[pl.ds(off, _B)],
                             idx_v.at[p, j, pl.ds(0, _B)], semi[p])

    def drain_idx(p):
        for j, h in enumerate(idx_hbms):
            pltpu.make_async_copy(h.at[pl.ds(0, _B)],
                                  idx_v.at[p, j, pl.ds(0, _B)],
                                  semi[p]).wait()

    def fire_rows(p):
        for j in range(_NT):
            pltpu.async_copy(node_hbm.at[idx_v.at[p, j, pl.ds(0, _B)]],
                             rows_v.at[p, pl.ds(j * _B, _B)], semr[p])

    def drain_rows(p):
        pltpu.make_async_copy(node_hbm.at[pl.ds(0, _NT * _B)],
                              rows_v.at[p], semr[p]).wait()

    def compute(p, loss):
        def up(v):
            return plsc.unpack(v, format=plsc.PackFormat.INTERLEAVED)

        def edge_body(e, loss2):
            et_e = idx_v[p, 4, pl.ds(e, _L)][0]
            accp = None
            accn = None
            for j in range(_D // (2 * _L)):
                sl = pl.ds(j * 2 * _L, 2 * _L)
                rv = rel_v[et_e, sl]
                d0, d1 = up(rows_v[p, e, sl] + rv - rows_v[p, _B + e, sl])
                dn0, dn1 = up(rows_v[p, 2 * _B + e, sl] + rv
                              - rows_v[p, 3 * _B + e, sl])
                if accp is None:
                    accp = d0 * d0 + d1 * d1
                    accn = dn0 * dn0 + dn1 * dn1
                else:
                    accp = accp + d0 * d0 + d1 * d1
                    accn = accn + dn0 * dn0 + dn1 * dn1
            pos = _sqrt_nr(jnp.sum(accp))
            neg = _sqrt_nr(jnp.sum(accn))
            return loss2 + jnp.maximum(pos - neg + 1.0, 0.0)

        return lax.fori_loop(0, _B, edge_body, loss, unroll=4)

    fire_idx(0, 0)
    fire_idx(1, 1)
    drain_idx(0)
    fire_rows(0)

    def pair_body(i, loss):
        drain_idx(1)
        fire_rows(1)
        drain_rows(0)

        @pl.when(2 * i + 2 < _NCHUNK)
        def _():
            fire_idx(2 * i + 2, 0)

        loss = compute(0, loss)

        @pl.when(2 * i + 2 < _NCHUNK)
        def _():
            drain_idx(0)
            fire_rows(0)

        @pl.when(2 * i + 3 < _NCHUNK)
        def _():
            fire_idx(2 * i + 3, 1)

        drain_rows(1)
        return compute(1, loss)

    total = lax.fori_loop(0, _NCHUNK // 2, pair_body, jnp.float32(0.0),
                          unroll=False)
    drain_rows(0)
    total = compute(0, total)

    acc_v[...] = jnp.where(lax.iota(jnp.int32, _L) == 0, total, 0.0)
    pltpu.sync_copy(acc_v, out_hbm.at[wid])


@functools.partial(jax.jit, static_argnames=())
def _loss_sc(node_embs, rel_weight, src, tgt, nsrc, ntgt, et):
    mesh = plsc.VectorSubcoreMesh(core_axis_name="c", subcore_axis_name="s")
    run = functools.partial(
        pl.kernel,
        mesh=mesh,
        compiler_params=pltpu.CompilerParams(needs_layout_passes=False,
                                             use_tc_tiling_on_sc=False),
        out_type=jax.ShapeDtypeStruct((_NW, _L), jnp.float32),
        scratch_types=[
            pltpu.VMEM((_NR, _D), jnp.bfloat16),
            pltpu.VMEM((2, 5, _D), jnp.int32),
            pltpu.VMEM((2, _NT * _B, _D), jnp.bfloat16),
            pltpu.VMEM((_L,), jnp.float32),
            pltpu.SemaphoreType.DMA,
            pltpu.SemaphoreType.DMA,
            pltpu.SemaphoreType.DMA,
            pltpu.SemaphoreType.DMA,
        ],
    )(_tile_body)
    return run(node_embs, rel_weight, src, tgt, nsrc, ntgt, et)


def kernel(node_embs, edge_index, edge_type, rel_weight):
    neg_edge_index = jax.random.randint(
        jax.random.key(42), edge_index.shape, 0, _E_SIZE, dtype=edge_index.dtype)
    partials = _loss_sc(
        node_embs.astype(jnp.bfloat16), rel_weight.astype(jnp.bfloat16),
        edge_index[0], edge_index[1],
        neg_edge_index[0], neg_edge_index[1],
        edge_type,
    )
    return jnp.sum(partials) / jnp.float32(_NE)


# speedup vs baseline: 1.0467x; 1.0467x over previous
import functools

import jax
import jax.numpy as jnp
from jax import lax
from jax.experimental import pallas as pl
from jax.experimental.pallas import tpu as pltpu
from jax.experimental.pallas import tpu_sc as plsc

_E_SIZE = 6884
_NE = 320000
_D = 128
_L = 16

_NC = 2
_NS = 16
_NW = _NC * _NS
_EPW = _NE // _NW
_B = 80
_NCHUNK = _EPW // _B
_NT = 5
_GRP = 8


def _sqrt_nr(x):
    i = lax.bitcast_convert_type(x, jnp.int32)
    y = lax.bitcast_convert_type(jnp.int32(0x5F3759DF) - (i >> 1), jnp.float32)
    for _ in range(2):
        y = y * (1.5 - 0.5 * x * y * y)
    return x * y


def _tile_body(node_hbm, rel_hbm, src_hbm, tgt_hbm, nsrc_hbm, ntgt_hbm, et_hbm,
               out_hbm,
               src_i, tgt_i, nsrc_i, ntgt_i, et_i,
               rows_v, acc_v, sem0, sem1):
    wid = lax.axis_index("s") * _NC + lax.axis_index("c")
    base = pl.multiple_of(wid * _EPW, 8)

    pltpu.sync_copy(src_hbm.at[pl.ds(base, _EPW)], src_i)
    pltpu.sync_copy(tgt_hbm.at[pl.ds(base, _EPW)], tgt_i)
    pltpu.sync_copy(nsrc_hbm.at[pl.ds(base, _EPW)], nsrc_i)
    pltpu.sync_copy(ntgt_hbm.at[pl.ds(base, _EPW)], ntgt_i)
    pltpu.sync_copy(et_hbm.at[pl.ds(base, _EPW)], et_i)

    sems = (sem0, sem1)

    def fire(c, p):
        off = pl.multiple_of(c * _B, 8)
        sem = sems[p]
        pltpu.async_copy(node_hbm.at[src_i.at[pl.ds(off, _B)]],
                         rows_v.at[p, pl.ds(0 * _B, _B)], sem)
        pltpu.async_copy(node_hbm.at[tgt_i.at[pl.ds(off, _B)]],
                         rows_v.at[p, pl.ds(1 * _B, _B)], sem)
        pltpu.async_copy(node_hbm.at[nsrc_i.at[pl.ds(off, _B)]],
                         rows_v.at[p, pl.ds(2 * _B, _B)], sem)
        pltpu.async_copy(node_hbm.at[ntgt_i.at[pl.ds(off, _B)]],
                         rows_v.at[p, pl.ds(3 * _B, _B)], sem)
        pltpu.async_copy(rel_hbm.at[et_i.at[pl.ds(off, _B)]],
                         rows_v.at[p, pl.ds(4 * _B, _B)], sem)

    def drain(p):
        pltpu.make_async_copy(node_hbm.at[pl.ds(0, _NT * _B)],
                              rows_v.at[p], sems[p]).wait()

    def compute(p, loss):
        def up(v):
            return plsc.unpack(v, format=plsc.PackFormat.INTERLEAVED)

        def edge_body(e, loss2):
            accp = None
            accn = None
            for j in range(_D // (2 * _L)):
                sl = pl.ds(j * 2 * _L, 2 * _L)
                rv = rows_v[p, 4 * _B + e, sl]
                d0, d1 = up(rows_v[p, e, sl] + rv - rows_v[p, _B + e, sl])
                dn0, dn1 = up(rows_v[p, 2 * _B + e, sl] + rv
                              - rows_v[p, 3 * _B + e, sl])
                if accp is None:
                    accp = d0 * d0 + d1 * d1
                    accn = dn0 * dn0 + dn1 * dn1
                else:
                    accp = accp + d0 * d0 + d1 * d1
                    accn = accn + dn0 * dn0 + dn1 * dn1
            pos = _sqrt_nr(jnp.sum(accp))
            neg = _sqrt_nr(jnp.sum(accn))
            return loss2 + jnp.maximum(pos - neg + 1.0, 0.0)

        return lax.fori_loop(0, _B, edge_body, loss, unroll=4)

    fire(0, 0)

    def pair_body(i, loss):
        fire(2 * i + 1, 1)
        drain(0)
        loss = compute(0, loss)
        fire(2 * i + 2, 0)
        drain(1)
        return compute(1, loss)

    total = lax.fori_loop(0, _NCHUNK // 2, pair_body, jnp.float32(0.0),
                          unroll=False)
    drain(0)
    total = compute(0, total)

    acc_v[...] = jnp.where(lax.iota(jnp.int32, _L) == 0, total, 0.0)
    pltpu.sync_copy(acc_v, out_hbm.at[wid])


@functools.partial(jax.jit, static_argnames=())
def _loss_sc(node_embs, rel_weight, src, tgt, nsrc, ntgt, et):
    mesh = plsc.VectorSubcoreMesh(core_axis_name="c", subcore_axis_name="s")
    run = functools.partial(
        pl.kernel,
        mesh=mesh,
        compiler_params=pltpu.CompilerParams(needs_layout_passes=False,
                                             use_tc_tiling_on_sc=False),
        out_type=jax.ShapeDtypeStruct((_NW, _L), jnp.float32),
        scratch_types=[
            pltpu.VMEM((_EPW,), jnp.int32),
            pltpu.VMEM((_EPW,), jnp.int32),
            pltpu.VMEM((_EPW,), jnp.int32),
            pltpu.VMEM((_EPW,), jnp.int32),
            pltpu.VMEM((_EPW,), jnp.int32),
            pltpu.VMEM((2, _NT * _B, _D), jnp.bfloat16),
            pltpu.VMEM((_L,), jnp.float32),
            pltpu.SemaphoreType.DMA,
            pltpu.SemaphoreType.DMA,
        ],
    )(_tile_body)
    return run(node_embs, rel_weight, src, tgt, nsrc, ntgt, et)


def kernel(node_embs, edge_index, edge_type, rel_weight):
    neg_edge_index = jax.random.randint(
        jax.random.key(42), edge_index.shape, 0, _E_SIZE, dtype=edge_index.dtype)
    partials = _loss_sc(
        node_embs.astype(jnp.bfloat16), rel_weight.astype(jnp.bfloat16),
        edge_index[0], edge_index[1],
        neg_edge_index[0], neg_edge_index[1],
        edge_type,
    )
    return jnp.sum(partials) / jnp.float32(_NE)
